# baseline (speedup 1.0000x reference)
import functools

import jax
import jax.numpy as jnp
from jax import lax
from jax.experimental import pallas as pl
from jax.experimental.pallas import tpu as pltpu
from jax.experimental.pallas import tpu_sc as plsc

_R = 8
_NC = 2
_NS = 16
_L = 16
_NPAD = 10240


def _sc_mesh():
    return plsc.VectorSubcoreMesh(core_axis_name="c", subcore_axis_name="s")


def _make_counts(n_nodes, n_edges, chunk):
    nr = n_nodes * _R
    ept = n_edges // (_NC * _NS)
    nch = ept // chunk
    assert ept * _NC * _NS == n_edges and nch * chunk == ept
    assert chunk % _L == 0 and nr % _NS == 0

    @functools.partial(
        pl.kernel,
        mesh=_sc_mesh(),
        out_type=jax.ShapeDtypeStruct((_NC * nr,), jnp.float32),
        scratch_types=[
            pltpu.VMEM_SHARED((nr,), jnp.float32),
            pltpu.VMEM((chunk,), jnp.int32),
            pltpu.VMEM((chunk,), jnp.int32),
            pltpu.VMEM((chunk,), jnp.int32),
            pltpu.VMEM((chunk,), jnp.float32),
        ],
    )
    def counts(dst, rel, out, c_sp, seg_v, dst_v, rel_v, ones_v):
        c = lax.axis_index("c")
        s = lax.axis_index("s")
        sl = nr // _NS

        def fill(val):
            def body(i, _):
                ones_v[pl.ds(i * _L, _L)] = jnp.full((_L,), val, jnp.float32)
                return 0
            lax.fori_loop(0, chunk // _L, body, 0)

        fill(0.0)
        off = s * sl
        done = 0
        while done < sl:
            step = min(chunk, sl - done)
            pltpu.sync_copy(ones_v.at[pl.ds(0, step)],
                            c_sp.at[pl.ds(off + done, step)])
            done += step
        fill(1.0)
        plsc.subcore_barrier()

        base = (c * _NS + s) * ept

        def do_chunk(g, _):
            b = base + g * chunk
            pltpu.sync_copy(dst.at[pl.ds(b, chunk)], dst_v)
            pltpu.sync_copy(rel.at[pl.ds(b, chunk)], rel_v)

            def segb(i, _):
                d = pl.ds(i * _L, _L)
                seg_v[d] = dst_v[d] * _R + rel_v[d]
                return 0

            lax.fori_loop(0, chunk // _L, segb, 0)
            pltpu.sync_copy(ones_v, c_sp.at[seg_v], add=True)
            return 0

        lax.fori_loop(0, nch, do_chunk, 0)
        plsc.subcore_barrier()
        done = 0
        while done < sl:
            step = min(chunk, sl - done)
            pltpu.sync_copy(c_sp.at[pl.ds(off + done, step)],
                            ones_v.at[pl.ds(0, step)])
            pltpu.sync_copy(ones_v.at[pl.ds(0, step)],
                            out.at[pl.ds(c * nr + off + done, step)])
            done += step

    return counts


def _make_agg(n_edges, width, chunk):
    ept = n_edges // (_NC * _NS)
    nch = ept // chunk
    assert nch * chunk == ept and chunk % _L == 0
    assert width % _L == 0 and _NPAD % _NS == 0
    wvecs = width // _L
    nsl = _NPAD // _NS
    nsets = 4

    scr = [pltpu.VMEM_SHARED((_NPAD, width), jnp.float32)]
    for _ in range(nsets):
        scr += [pltpu.VMEM((chunk,), jnp.int32)] * 3
        scr += [pltpu.VMEM((chunk,), jnp.float32),
                pltpu.VMEM((chunk, width), jnp.float32)]
    scr += [pltpu.SemaphoreType.DMA] * (3 * nsets)

    @functools.partial(
        pl.kernel,
        mesh=_sc_mesh(),
        out_type=jax.ShapeDtypeStruct((_NC, _NPAD, width), jnp.float32),
        scratch_types=scr,
        compiler_params=pltpu.CompilerParams(needs_layout_passes=False,
                                             use_tc_tiling_on_sc=False),
    )
    def agg(gidx, seg, dsti, cinv, y, out, *s):
        acc_sp = s[0]
        sets = []
        k = 1
        for _ in range(nsets):
            sets.append(dict(gidx=s[k], seg=s[k + 1], dsti=s[k + 2],
                             w=s[k + 3], rows=s[k + 4]))
            k += 5
        sem_ld = s[k:k + nsets]
        sem_gt = s[k + nsets:k + 2 * nsets]
        sem_sc = s[k + 2 * nsets:k + 3 * nsets]

        c = lax.axis_index("c")
        tid = lax.axis_index("s")
        rows0 = sets[0]["rows"]

        zrows = 128
        assert nsl % zrows == 0

        def zbody(i, _):
            for j in range(wvecs):
                rows0[i, pl.ds(j * _L, _L)] = jnp.zeros((_L,), jnp.float32)
            return 0

        lax.fori_loop(0, zrows, zbody, 0)
        for q in range(nsl // zrows):
            pltpu.sync_copy(
                rows0.at[pl.ds(0, zrows), :],
                acc_sp.at[pl.ds(tid * nsl + q * zrows, zrows), :])
        plsc.subcore_barrier()

        base = (c * _NS + tid) * ept

        def issue_loads(g):
            u = g % nsets
            S = sets[u]
            b = base + g * chunk
            pltpu.async_copy(gidx.at[pl.ds(b, chunk)], S["gidx"], sem_ld[u])
            pltpu.async_copy(seg.at[pl.ds(b, chunk)], S["seg"], sem_ld[u])
            pltpu.async_copy(dsti.at[pl.ds(b, chunk)], S["dsti"], sem_ld[u])

        def wait_loads(g):
            u = g % nsets
            S = sets[u]
            for d in (S["gidx"], S["seg"], S["dsti"]):
                pltpu.make_async_copy(gidx.at[pl.ds(0, chunk)], d,
                                      sem_ld[u]).wait()

        def issue_gathers(g):
            u = g % nsets
            S = sets[u]
            pltpu.async_copy(cinv.at[S["seg"]], S["w"], sem_gt[u])
            pltpu.async_copy(y.at[S["gidx"]], S["rows"], sem_gt[u])

        def wait_gathers(g):
            u = g % nsets
            S = sets[u]
            pltpu.make_async_copy(cinv.at[S["seg"]], S["w"], sem_gt[u]).wait()
            pltpu.make_async_copy(y.at[S["gidx"]], S["rows"], sem_gt[u]).wait()

        def scale(g):
            S = sets[g % nsets]

            def body(q, _):
                w16 = S["w"][pl.ds(q * _L, _L)]
                for l in range(_L):
                    wv = jnp.take(w16, jnp.full((_L,), l, jnp.int32))
                    i = q * _L + l
                    for j in range(wvecs):
                        S["rows"][i, pl.ds(j * _L, _L)] = (
                            S["rows"][i, pl.ds(j * _L, _L)] * wv)
                return 0

            lax.fori_loop(0, chunk // _L, body, 0)

        def issue_scatter(g):
            u = g % nsets
            S = sets[u]
            pltpu.async_copy(S["rows"], acc_sp.at[S["dsti"]], sem_sc[u],
                             add=True)

        def wait_scatter(g):
            u = g % nsets
            S = sets[u]
            pltpu.make_async_copy(S["rows"], acc_sp.at[S["dsti"]],
                                  sem_sc[u]).wait()

        for g in range(min(3, nch)):
            issue_loads(g)
        wait_loads(0)
        issue_gathers(0)
        for g in range(nch):
            if g + 1 < nch:
                wait_loads(g + 1)
                issue_gathers(g + 1)
            if g - 1 >= 0:
                wait_scatter(g - 1)
            if g + 3 < nch:
                issue_loads(g + 3)
            wait_gathers(g)
            scale(g)
            issue_scatter(g)
        wait_scatter(nch - 1)
        plsc.subcore_barrier()
        piece = nsl
        while piece > chunk:
            piece //= 2
        for q in range(nsl // piece):
            pltpu.sync_copy(
                acc_sp.at[pl.ds(tid * nsl + q * piece, piece), :],
                rows0.at[pl.ds(0, piece), :])
            pltpu.sync_copy(
                rows0.at[pl.ds(0, piece), :],
                out.at[c, pl.ds(tid * nsl + q * piece, piece), :])

    return agg


def _edge_prep_tc(edge_index, edge_weight, blk):
    e = edge_weight.shape[0]

    def kern(ei_ref, ew_ref, g_ref, s_ref, d_ref):
        src = ei_ref[0]
        dst = ei_ref[1]
        et = ew_ref[...]
        g_ref[...] = src * _R + et
        s_ref[...] = dst * _R + et
        d_ref[...] = dst

    return pl.pallas_call(
        kern,
        out_shape=[jax.ShapeDtypeStruct((e,), jnp.int32)] * 3,
    )(edge_index, edge_weight)


def _project_tc(x, w, block):
    n, ic = x.shape
    oc = w.shape[1]
    grid = n // block
    ocr = oc // 128

    def kern(x_ref, w_ref, y_ref):
        z = jnp.dot(x_ref[...], w_ref[...],
                    preferred_element_type=jnp.float32)
        y_ref[...] = z.reshape(block * ocr, 128)

    return pl.pallas_call(
        kern,
        grid=(grid,),
        in_specs=[
            pl.BlockSpec((block, ic), lambda i: (i, 0)),
            pl.BlockSpec((ic, oc), lambda i: (0, 0)),
        ],
        out_specs=pl.BlockSpec((block * ocr, 128), lambda i: (i, 0)),
        out_shape=jax.ShapeDtypeStruct((n * ocr, 128), jnp.float32),
    )(x, w)


def _cinv_tc(c_part, nr):
    rows = nr // 128

    def kern(c_ref, ci_ref):
        tot = c_ref[0] + c_ref[1]
        ci_ref[...] = 1.0 / jnp.maximum(tot, 1.0)

    ci = pl.pallas_call(
        kern,
        in_specs=[pl.BlockSpec((_NC, rows, 128), lambda: (0, 0, 0))],
        out_specs=pl.BlockSpec((rows, 128), lambda: (0, 0)),
        out_shape=jax.ShapeDtypeStruct((rows, 128), jnp.float32),
    )(c_part.reshape(_NC, rows, 128))
    return ci.reshape(nr)


def _layer1_tc(acc, x, root1, bias1, w2flat, root2, bias2, block):
    n = x.shape[0]
    ic = x.shape[1]
    oc2 = w2flat.shape[1]
    grid = n // block

    def kern(a_ref, x_ref, r1_ref, b1_ref, w2_ref, r2_ref, b2_ref,
             y2_ref, rt2_ref):
        z = (a_ref[0] + a_ref[1]
             + jnp.dot(x_ref[...], r1_ref[...],
                       preferred_element_type=jnp.float32)
             + b1_ref[...])
        gi = jax.nn.sigmoid(z[:, 0:16])
        gc = jnp.tanh(z[:, 16:32])
        go = jax.nn.sigmoid(z[:, 32:48])
        h1 = go * jnp.tanh(gi * gc)
        y2 = jnp.dot(h1, w2_ref[...], preferred_element_type=jnp.float32)
        y2_ref[...] = y2.reshape(block * (oc2 // 128), 128)
        rt2_ref[...] = (jnp.dot(h1, r2_ref[...],
                                preferred_element_type=jnp.float32)
                        + b2_ref[...])

    return pl.pallas_call(
        kern,
        grid=(grid,),
        in_specs=[
            pl.BlockSpec((_NC, block, 48), lambda i: (0, i, 0)),
            pl.BlockSpec((block, ic), lambda i: (i, 0)),
            pl.BlockSpec((ic, 48), lambda i: (0, 0)),
            pl.BlockSpec((1, 48), lambda i: (0, 0)),
            pl.BlockSpec((16, oc2), lambda i: (0, 0)),
            pl.BlockSpec((16, 24), lambda i: (0, 0)),
            pl.BlockSpec((1, 24), lambda i: (0, 0)),
        ],
        out_specs=[
            pl.BlockSpec((block * (oc2 // 128), 128), lambda i: (i, 0)),
            pl.BlockSpec((block, 24), lambda i: (i, 0)),
        ],
        out_shape=[
            jax.ShapeDtypeStruct((n * (oc2 // 128), 128), jnp.float32),
            jax.ShapeDtypeStruct((n, 24), jnp.float32),
        ],
    )(acc, x, root1, bias1, w2flat, root2, bias2)


def _final_tc(acc2, rt2, lin_w, lin_b, block):
    n = rt2.shape[0]
    grid = n // block
    ncls = lin_w.shape[1]

    def kern(a_ref, rt_ref, w_ref, b_ref, o_ref):
        z = a_ref[0, :, 0:24] + a_ref[1, :, 0:24] + rt_ref[...]
        gi = jax.nn.sigmoid(z[:, 0:8])
        gc = jnp.tanh(z[:, 8:16])
        go = jax.nn.sigmoid(z[:, 16:24])
        h2 = go * jnp.tanh(gi * gc)
        h2 = jnp.maximum(h2, 0.0)
        logits = jnp.dot(h2, w_ref[...],
                         preferred_element_type=jnp.float32) + b_ref[...]
        m = jnp.max(logits, axis=1, keepdims=True)
        e = logits - m
        o_ref[...] = e - jnp.log(jnp.sum(jnp.exp(e), axis=1, keepdims=True))

    return pl.pallas_call(
        kern,
        grid=(grid,),
        in_specs=[
            pl.BlockSpec((_NC, block, 32), lambda i: (0, i, 0)),
            pl.BlockSpec((block, 24), lambda i: (i, 0)),
            pl.BlockSpec((8, ncls), lambda i: (0, 0)),
            pl.BlockSpec((1, ncls), lambda i: (0, 0)),
        ],
        out_specs=pl.BlockSpec((block, ncls), lambda i: (i, 0)),
        out_shape=jax.ShapeDtypeStruct((n, ncls), jnp.float32),
    )(acc2, rt2, lin_w, lin_b)


def _conv_w(p):
    return jnp.einsum("rb,bio->rio", p["comp"], p["basis"])


def kernel(x, edge_index, edge_weight, params):
    n, ic = x.shape
    e = edge_index.shape[1]
    nr = n * _R
    p1, p2 = params["l1"], params["l2"]
    block = 1000
    chunk = 2000

    gates = ("x_i", "x_c", "x_o")
    w1 = jnp.concatenate([_conv_w(p1[g]) for g in gates], axis=-1)
    w1flat = w1.transpose(1, 0, 2).reshape(ic, _R * 48)
    root1 = jnp.concatenate([p1[g]["root"] for g in gates], axis=-1)
    bias1 = jnp.concatenate(
        [p1[g]["bias"] + p1[h]["bias"]
         for g, h in zip(gates, ("h_i", "h_c", "h_o"))])[None, :]
    w2 = jnp.concatenate([_conv_w(p2[g]) for g in gates], axis=-1)
    w2p = jnp.pad(w2, ((0, 0), (0, 0), (0, 8)))
    w2flat = w2p.transpose(1, 0, 2).reshape(16, _R * 32)
    root2 = jnp.concatenate([p2[g]["root"] for g in gates], axis=-1)
    bias2 = jnp.concatenate(
        [p2[g]["bias"] + p2[h]["bias"]
         for g, h in zip(gates, ("h_i", "h_c", "h_o"))])[None, :]

    ei32 = edge_index.astype(jnp.int32)
    ew32 = edge_weight.astype(jnp.int32)
    c_part = _make_counts(n, e, chunk)(ei32[1], ew32)
    gidx, seg, dsti = _edge_prep_tc(ei32, ew32, 20000)
    y1_ = _project_tc(x, w1flat, block)
    cinv = _cinv_tc(c_part, nr)
    y1 = y1_.reshape(nr, 48)
    acc1 = _make_agg(e, 48, 400)(gidx, seg, dsti, cinv, y1)

    y2_, rt2 = _layer1_tc(acc1, x, root1, bias1, w2flat, root2, bias2, block)
    y2 = y2_.reshape(nr, 32)
    acc2 = _make_agg(e, 32, 400)(gidx, seg, dsti, cinv, y2)

    return _final_tc(acc2, rt2, params["lin_w"], params["lin_b"][None, :],
                     block)


# speedup vs baseline: 1.0886x; 1.0886x over previous
import functools

import jax
import jax.numpy as jnp
from jax import lax
from jax.experimental import pallas as pl
from jax.experimental.pallas import tpu as pltpu
from jax.experimental.pallas import tpu_sc as plsc

_R = 8
_NC = 2
_NS = 16
_L = 16
_NPAD = 10240


def _sc_mesh():
    return plsc.VectorSubcoreMesh(core_axis_name="c", subcore_axis_name="s")


def _make_counts(n_nodes, n_edges, chunk):
    nr = n_nodes * _R
    ept = n_edges // (_NC * _NS)
    nch = ept // chunk
    assert ept * _NC * _NS == n_edges and nch * chunk == ept
    assert chunk % _L == 0 and nr % _NS == 0

    @functools.partial(
        pl.kernel,
        mesh=_sc_mesh(),
        out_type=jax.ShapeDtypeStruct((_NC * nr,), jnp.float32),
        scratch_types=[
            pltpu.VMEM_SHARED((nr,), jnp.float32),
            pltpu.VMEM((chunk,), jnp.int32),
            pltpu.VMEM((chunk,), jnp.float32),
        ],
    )
    def counts(seg, out, c_sp, seg_v, ones_v):
        c = lax.axis_index("c")
        s = lax.axis_index("s")
        sl = nr // _NS

        def fill(val):
            def body(i, _):
                ones_v[pl.ds(i * _L, _L)] = jnp.full((_L,), val, jnp.float32)
                return 0
            lax.fori_loop(0, chunk // _L, body, 0)

        fill(0.0)
        off = s * sl
        done = 0
        while done < sl:
            step = min(chunk, sl - done)
            pltpu.sync_copy(ones_v.at[pl.ds(0, step)],
                            c_sp.at[pl.ds(off + done, step)])
            done += step
        fill(1.0)
        plsc.subcore_barrier()

        base = (c * _NS + s) * ept

        def do_chunk(g, _):
            b = base + g * chunk
            pltpu.sync_copy(seg.at[pl.ds(b, chunk)], seg_v)
            pltpu.sync_copy(ones_v, c_sp.at[seg_v], add=True)
            return 0

        lax.fori_loop(0, nch, do_chunk, 0)
        plsc.subcore_barrier()
        done = 0
        while done < sl:
            step = min(chunk, sl - done)
            pltpu.sync_copy(c_sp.at[pl.ds(off + done, step)],
                            ones_v.at[pl.ds(0, step)])
            pltpu.sync_copy(ones_v.at[pl.ds(0, step)],
                            out.at[pl.ds(c * nr + off + done, step)])
            done += step

    return counts


def _make_agg(n_edges, width, chunk):
    ept = n_edges // (_NC * _NS)
    nch = ept // chunk
    assert nch * chunk == ept and chunk % _L == 0
    assert width % _L == 0 and _NPAD % _NS == 0
    wvecs = width // _L
    nsl = _NPAD // _NS
    nsets = 4

    scr = [pltpu.VMEM_SHARED((_NPAD, width), jnp.float32)]
    for _ in range(nsets):
        scr += [pltpu.VMEM((chunk,), jnp.int32)] * 3
        scr += [pltpu.VMEM((chunk,), jnp.float32),
                pltpu.VMEM((chunk, width), jnp.float32)]
    scr += [pltpu.SemaphoreType.DMA] * (3 * nsets)

    @functools.partial(
        pl.kernel,
        mesh=_sc_mesh(),
        out_type=jax.ShapeDtypeStruct((_NC, _NPAD, width), jnp.float32),
        scratch_types=scr,
        compiler_params=pltpu.CompilerParams(needs_layout_passes=False,
                                             use_tc_tiling_on_sc=False),
    )
    def agg(gidx, seg, dsti, cinv, y, out, *s):
        acc_sp = s[0]
        sets = []
        k = 1
        for _ in range(nsets):
            sets.append(dict(gidx=s[k], seg=s[k + 1], dsti=s[k + 2],
                             w=s[k + 3], rows=s[k + 4]))
            k += 5
        sem_ld = s[k:k + nsets]
        sem_gt = s[k + nsets:k + 2 * nsets]
        sem_sc = s[k + 2 * nsets:k + 3 * nsets]

        c = lax.axis_index("c")
        tid = lax.axis_index("s")
        rows0 = sets[0]["rows"]

        zrows = 128
        assert nsl % zrows == 0

        def zbody(i, _):
            for j in range(wvecs):
                rows0[i, pl.ds(j * _L, _L)] = jnp.zeros((_L,), jnp.float32)
            return 0

        lax.fori_loop(0, zrows, zbody, 0)
        for q in range(nsl // zrows):
            pltpu.sync_copy(
                rows0.at[pl.ds(0, zrows), :],
                acc_sp.at[pl.ds(tid * nsl + q * zrows, zrows), :])
        plsc.subcore_barrier()

        base = (c * _NS + tid) * ept

        def issue_loads(g):
            u = g % nsets
            S = sets[u]
            b = base + g * chunk
            pltpu.async_copy(gidx.at[pl.ds(b, chunk)], S["gidx"], sem_ld[u])
            pltpu.async_copy(seg.at[pl.ds(b, chunk)], S["seg"], sem_ld[u])
            pltpu.async_copy(dsti.at[pl.ds(b, chunk)], S["dsti"], sem_ld[u])

        def wait_loads(g):
            u = g % nsets
            S = sets[u]
            for d in (S["gidx"], S["seg"], S["dsti"]):
                pltpu.make_async_copy(gidx.at[pl.ds(0, chunk)], d,
                                      sem_ld[u]).wait()

        def issue_gathers(g):
            u = g % nsets
            S = sets[u]
            pltpu.async_copy(cinv.at[S["seg"]], S["w"], sem_gt[u])
            pltpu.async_copy(y.at[S["gidx"]], S["rows"], sem_gt[u])

        def wait_gathers(g):
            u = g % nsets
            S = sets[u]
            pltpu.make_async_copy(cinv.at[S["seg"]], S["w"], sem_gt[u]).wait()
            pltpu.make_async_copy(y.at[S["gidx"]], S["rows"], sem_gt[u]).wait()

        def scale(g):
            S = sets[g % nsets]

            def body(q, _):
                w16 = S["w"][pl.ds(q * _L, _L)]
                for l in range(_L):
                    wv = jnp.take(w16, jnp.full((_L,), l, jnp.int32))
                    i = q * _L + l
                    for j in range(wvecs):
                        S["rows"][i, pl.ds(j * _L, _L)] = (
                            S["rows"][i, pl.ds(j * _L, _L)] * wv)
                return 0

            lax.fori_loop(0, chunk // _L, body, 0)

        def issue_scatter(g):
            u = g % nsets
            S = sets[u]
            pltpu.async_copy(S["rows"], acc_sp.at[S["dsti"]], sem_sc[u],
                             add=True)

        def wait_scatter(g):
            u = g % nsets
            S = sets[u]
            pltpu.make_async_copy(S["rows"], acc_sp.at[S["dsti"]],
                                  sem_sc[u]).wait()

        for g in range(min(3, nch)):
            issue_loads(g)
        wait_loads(0)
        issue_gathers(0)
        for g in range(nch):
            if g + 1 < nch:
                wait_loads(g + 1)
                issue_gathers(g + 1)
            if g - 1 >= 0:
                wait_scatter(g - 1)
            if g + 3 < nch:
                issue_loads(g + 3)
            wait_gathers(g)
            scale(g)
            issue_scatter(g)
        wait_scatter(nch - 1)
        plsc.subcore_barrier()
        piece = nsl
        while piece > chunk:
            piece //= 2
        for q in range(nsl // piece):
            pltpu.sync_copy(
                acc_sp.at[pl.ds(tid * nsl + q * piece, piece), :],
                rows0.at[pl.ds(0, piece), :])
            pltpu.sync_copy(
                rows0.at[pl.ds(0, piece), :],
                out.at[c, pl.ds(tid * nsl + q * piece, piece), :])

    return agg


def _edge_prep_tc(edge_index, edge_weight, blk):
    e = edge_weight.shape[0]

    def kern(ei_ref, ew_ref, g_ref, s_ref, d_ref):
        src = ei_ref[0]
        dst = ei_ref[1]
        et = ew_ref[...]
        g_ref[...] = src * _R + et
        s_ref[...] = dst * _R + et
        d_ref[...] = dst

    return pl.pallas_call(
        kern,
        out_shape=[jax.ShapeDtypeStruct((e,), jnp.int32)] * 3,
    )(edge_index, edge_weight)


def _project_tc(x, w, block):
    n, ic = x.shape
    oc = w.shape[1]
    grid = n // block
    ocr = oc // 128

    def kern(x_ref, w_ref, y_ref):
        z = jnp.dot(x_ref[...], w_ref[...],
                    preferred_element_type=jnp.float32)
        y_ref[...] = z.reshape(block * ocr, 128)

    return pl.pallas_call(
        kern,
        grid=(grid,),
        in_specs=[
            pl.BlockSpec((block, ic), lambda i: (i, 0)),
            pl.BlockSpec((ic, oc), lambda i: (0, 0)),
        ],
        out_specs=pl.BlockSpec((block * ocr, 128), lambda i: (i, 0)),
        out_shape=jax.ShapeDtypeStruct((n * ocr, 128), jnp.float32),
    )(x, w)


def _cinv_tc(c_part, nr):
    rows = nr // 128

    def kern(c_ref, ci_ref):
        tot = c_ref[0] + c_ref[1]
        ci_ref[...] = 1.0 / jnp.maximum(tot, 1.0)

    ci = pl.pallas_call(
        kern,
        in_specs=[pl.BlockSpec((_NC, rows, 128), lambda: (0, 0, 0))],
        out_specs=pl.BlockSpec((rows, 128), lambda: (0, 0)),
        out_shape=jax.ShapeDtypeStruct((rows, 128), jnp.float32),
    )(c_part.reshape(_NC, rows, 128))
    return ci.reshape(nr)


def _layer1_tc(acc, x, root1, bias1, w2flat, root2, bias2, block):
    n = x.shape[0]
    ic = x.shape[1]
    oc2 = w2flat.shape[1]
    grid = n // block

    def kern(a_ref, x_ref, r1_ref, b1_ref, w2_ref, r2_ref, b2_ref,
             y2_ref, rt2_ref):
        z = (a_ref[0] + a_ref[1]
             + jnp.dot(x_ref[...], r1_ref[...],
                       preferred_element_type=jnp.float32)
             + b1_ref[...])
        gi = jax.nn.sigmoid(z[:, 0:16])
        gc = jnp.tanh(z[:, 16:32])
        go = jax.nn.sigmoid(z[:, 32:48])
        h1 = go * jnp.tanh(gi * gc)
        y2 = jnp.dot(h1, w2_ref[...], preferred_element_type=jnp.float32)
        y2_ref[...] = y2.reshape(block * (oc2 // 128), 128)
        rt2_ref[...] = (jnp.dot(h1, r2_ref[...],
                                preferred_element_type=jnp.float32)
                        + b2_ref[...])

    return pl.pallas_call(
        kern,
        grid=(grid,),
        in_specs=[
            pl.BlockSpec((_NC, block, 48), lambda i: (0, i, 0)),
            pl.BlockSpec((block, ic), lambda i: (i, 0)),
            pl.BlockSpec((ic, 48), lambda i: (0, 0)),
            pl.BlockSpec((1, 48), lambda i: (0, 0)),
            pl.BlockSpec((16, oc2), lambda i: (0, 0)),
            pl.BlockSpec((16, 24), lambda i: (0, 0)),
            pl.BlockSpec((1, 24), lambda i: (0, 0)),
        ],
        out_specs=[
            pl.BlockSpec((block * (oc2 // 128), 128), lambda i: (i, 0)),
            pl.BlockSpec((block, 24), lambda i: (i, 0)),
        ],
        out_shape=[
            jax.ShapeDtypeStruct((n * (oc2 // 128), 128), jnp.float32),
            jax.ShapeDtypeStruct((n, 24), jnp.float32),
        ],
    )(acc, x, root1, bias1, w2flat, root2, bias2)


def _final_tc(acc2, rt2, lin_w, lin_b, block):
    n = rt2.shape[0]
    grid = n // block
    ncls = lin_w.shape[1]

    def kern(a_ref, rt_ref, w_ref, b_ref, o_ref):
        z = a_ref[0, :, 0:24] + a_ref[1, :, 0:24] + rt_ref[...]
        gi = jax.nn.sigmoid(z[:, 0:8])
        gc = jnp.tanh(z[:, 8:16])
        go = jax.nn.sigmoid(z[:, 16:24])
        h2 = go * jnp.tanh(gi * gc)
        h2 = jnp.maximum(h2, 0.0)
        logits = jnp.dot(h2, w_ref[...],
                         preferred_element_type=jnp.float32) + b_ref[...]
        m = jnp.max(logits, axis=1, keepdims=True)
        e = logits - m
        o_ref[...] = e - jnp.log(jnp.sum(jnp.exp(e), axis=1, keepdims=True))

    return pl.pallas_call(
        kern,
        grid=(grid,),
        in_specs=[
            pl.BlockSpec((_NC, block, 32), lambda i: (0, i, 0)),
            pl.BlockSpec((block, 24), lambda i: (i, 0)),
            pl.BlockSpec((8, ncls), lambda i: (0, 0)),
            pl.BlockSpec((1, ncls), lambda i: (0, 0)),
        ],
        out_specs=pl.BlockSpec((block, ncls), lambda i: (i, 0)),
        out_shape=jax.ShapeDtypeStruct((n, ncls), jnp.float32),
    )(acc2, rt2, lin_w, lin_b)


def _conv_w(p):
    return jnp.einsum("rb,bio->rio", p["comp"], p["basis"])


def kernel(x, edge_index, edge_weight, params):
    n, ic = x.shape
    e = edge_index.shape[1]
    nr = n * _R
    p1, p2 = params["l1"], params["l2"]
    block = 1000
    chunk = 2000

    gates = ("x_i", "x_c", "x_o")
    w1 = jnp.concatenate([_conv_w(p1[g]) for g in gates], axis=-1)
    w1flat = w1.transpose(1, 0, 2).reshape(ic, _R * 48)
    root1 = jnp.concatenate([p1[g]["root"] for g in gates], axis=-1)
    bias1 = jnp.concatenate(
        [p1[g]["bias"] + p1[h]["bias"]
         for g, h in zip(gates, ("h_i", "h_c", "h_o"))])[None, :]
    w2 = jnp.concatenate([_conv_w(p2[g]) for g in gates], axis=-1)
    w2p = jnp.pad(w2, ((0, 0), (0, 0), (0, 8)))
    w2flat = w2p.transpose(1, 0, 2).reshape(16, _R * 32)
    root2 = jnp.concatenate([p2[g]["root"] for g in gates], axis=-1)
    bias2 = jnp.concatenate(
        [p2[g]["bias"] + p2[h]["bias"]
         for g, h in zip(gates, ("h_i", "h_c", "h_o"))])[None, :]

    gidx, seg, dsti = _edge_prep_tc(edge_index.astype(jnp.int32),
                                    edge_weight.astype(jnp.int32), 20000)
    c_part = _make_counts(n, e, chunk)(seg)
    y1_ = _project_tc(x, w1flat, block)
    cinv = _cinv_tc(c_part, nr)
    y1 = y1_.reshape(nr, 48)
    acc1 = _make_agg(e, 48, 400)(gidx, seg, dsti, cinv, y1)

    y2_, rt2 = _layer1_tc(acc1, x, root1, bias1, w2flat, root2, bias2, block)
    y2 = y2_.reshape(nr, 32)
    acc2 = _make_agg(e, 32, 400)(gidx, seg, dsti, cinv, y2)

    return _final_tc(acc2, rt2, params["lin_w"], params["lin_b"][None, :],
                     block)
